# baseline (speedup 1.0000x reference)
import functools

import jax
import jax.numpy as jnp
from jax import lax
from jax.experimental import pallas as pl
from jax.experimental.pallas import tpu as pltpu
from jax.experimental.pallas import tpu_sc as plsc

B, C_IN, L = 16, 128, 4096
HID, D, K = 384, 256, 8192
L2, L4 = L // 2, L // 4
NTOK = B * L4

TT = 1024
KK = 1024
NT_T = NTOK // TT
NT_K = K // KK

_BF = jnp.bfloat16
_F32 = jnp.float32


def _conv_body(r_ref, w1_ref, b1_ref, w2_ref, b2_ref, out_ref):
    r = r_ref[0]
    zs128 = jnp.zeros((1, C_IN), _BF)
    xm1 = jnp.concatenate([zs128, r[:-1, 3 * C_IN:4 * C_IN]], axis=0)
    a_e = jnp.concatenate([xm1, r[:, 0:3 * C_IN]], axis=1)
    xp4 = jnp.concatenate([r[1:, 0:C_IN], zs128], axis=0)
    a_o = jnp.concatenate([r[:, C_IN:4 * C_IN], xp4], axis=1)
    w1 = w1_ref[...]
    b1 = b1_ref[...]
    z1e = jnp.dot(a_e, w1, preferred_element_type=_F32) + b1
    z1o = jnp.dot(a_o, w1, preferred_element_type=_F32) + b1
    z1e_b = z1e.astype(_BF)
    z1o_b = z1o.astype(_BF)
    zs384 = jnp.zeros((1, HID), _BF)
    sd_o = jnp.concatenate([zs384, z1o_b[:-1]], axis=0)
    su_e = jnp.concatenate([z1e_b[1:], zs384], axis=0)
    w2 = w2_ref[...]
    ze = jnp.dot(sd_o, w2[0:HID], preferred_element_type=_F32)
    ze = ze + jnp.dot(z1e_b, w2[HID:2 * HID], preferred_element_type=_F32)
    ze = ze + jnp.dot(z1o_b, w2[2 * HID:3 * HID], preferred_element_type=_F32)
    ze = ze + jnp.dot(su_e, w2[3 * HID:4 * HID], preferred_element_type=_F32)
    ze = ze + b2_ref[...]
    out_ref[0] = ze


def _conv_call(r, w1c, b1, w2c, b2):
    return pl.pallas_call(
        _conv_body,
        grid=(B,),
        in_specs=[
            pl.BlockSpec((1, L4, 4 * C_IN), lambda b: (b, 0, 0)),
            pl.BlockSpec((4 * C_IN, HID), lambda b: (0, 0)),
            pl.BlockSpec((1, HID), lambda b: (0, 0)),
            pl.BlockSpec((4 * HID, D), lambda b: (0, 0)),
            pl.BlockSpec((1, D), lambda b: (0, 0)),
        ],
        out_specs=pl.BlockSpec((1, L4, D), lambda b: (b, 0, 0)),
        out_shape=jax.ShapeDtypeStruct((B, L4, D), jnp.float32),
    )(r, w1c, b1, w2c, b2)


def _csq_body(cb_ref, out_ref):
    cb = cb_ref[...]
    s = jnp.sum(cb * cb, axis=1)
    out_ref[...] = s.reshape(1, KK)


def _csq_call(cb):
    return pl.pallas_call(
        _csq_body,
        grid=(NT_K,),
        in_specs=[pl.BlockSpec((KK, D), lambda j: (j, 0))],
        out_specs=pl.BlockSpec((1, KK), lambda j: (0, j)),
        out_shape=jax.ShapeDtypeStruct((1, K), jnp.float32),
    )(cb)


_TILES_PER_CHUNK = 4096 // KK


def _vq_body(z_ref, cbt_ref, csq_ref, idx_ref, zsq_s, zm2_s, cmin_s, cidx_s, gmin_s, gidx_s):
    j = pl.program_id(1)

    @pl.when(j == 0)
    def _init():
        z = z_ref[...]
        zsq_s[...] = jnp.sum(z * z, axis=1, keepdims=True)
        zm2_s[...] = (-2.0 * z).astype(_BF)
        gmin_s[...] = jnp.full((TT, 1), jnp.inf, jnp.float32)
        gidx_s[...] = jnp.zeros((TT, 1), jnp.int32)

    @pl.when(j % _TILES_PER_CHUNK == 0)
    def _chunk_init():
        cmin_s[...] = jnp.full((TT, 1), jnp.inf, jnp.float32)
        cidx_s[...] = jnp.zeros((TT, 1), jnp.int32)

    mm2 = jnp.dot(zm2_s[...], cbt_ref[...], preferred_element_type=_F32)
    dist = (zsq_s[...] + mm2) + csq_ref[...]
    tmin = jnp.min(dist, axis=1, keepdims=True)
    lanes = lax.broadcasted_iota(jnp.int32, (TT, KK), 1)
    tidx = jnp.min(jnp.where(dist == tmin, lanes, K), axis=1, keepdims=True)
    tidx = tidx + j * KK
    better = tmin < cmin_s[...]
    cmin_s[...] = jnp.where(better, tmin, cmin_s[...])
    cidx_s[...] = jnp.where(better, tidx, cidx_s[...])

    @pl.when(j % _TILES_PER_CHUNK == _TILES_PER_CHUNK - 1)
    def _commit():
        take = cmin_s[...] < gmin_s[...]
        gmin_s[...] = jnp.where(take, cmin_s[...], gmin_s[...]).astype(_BF).astype(_F32)
        gidx_s[...] = jnp.where(take, cidx_s[...], gidx_s[...])

    @pl.when(j == NT_K - 1)
    def _out():
        idx_ref[...] = gidx_s[...]


def _vq_call(z_flat, cbt, csq):
    return pl.pallas_call(
        _vq_body,
        grid=(NT_T, NT_K),
        in_specs=[
            pl.BlockSpec((TT, D), lambda i, j: (i, 0)),
            pl.BlockSpec((D, KK), lambda i, j: (0, j)),
            pl.BlockSpec((1, KK), lambda i, j: (0, j)),
        ],
        out_specs=pl.BlockSpec((TT, 1), lambda i, j: (i, 0)),
        out_shape=jax.ShapeDtypeStruct((NTOK, 1), jnp.int32),
        scratch_shapes=[
            pltpu.VMEM((TT, 1), jnp.float32),
            pltpu.VMEM((TT, D), _BF),
            pltpu.VMEM((TT, 1), jnp.float32),
            pltpu.VMEM((TT, 1), jnp.int32),
            pltpu.VMEM((TT, 1), jnp.float32),
            pltpu.VMEM((TT, 1), jnp.int32),
        ],
    )(z_flat, cbt, csq)


_SC_NW = 32
_ROWS_PER_W = NTOK // _SC_NW
_CHUNK = 128


_NCHUNK = _ROWS_PER_W // _CHUNK


_NBUF = 3


def _gather_body(cb_hbm, idx_hbm, out_hbm, idx_v,
                 rows_a, rows_b, rows_c,
                 sg0, sg1, sg2, ss0, ss1, ss2):
    wid = lax.axis_index("s") * 2 + lax.axis_index("c")
    base = wid * _ROWS_PER_W
    pltpu.sync_copy(idx_hbm.at[pl.ds(base, _ROWS_PER_W)], idx_v)
    bufs = (rows_a, rows_b, rows_c)
    sem_g = (sg0, sg1, sg2)
    sem_s = (ss0, ss1, ss2)

    def start_gather(c):
        return pltpu.async_copy(
            cb_hbm.at[idx_v.at[pl.ds(c * _CHUNK, _CHUNK)]],
            bufs[c % _NBUF], sem_g[c % _NBUF],
        )

    gathers = [None] * _NCHUNK
    scatters = [None] * _NCHUNK
    for c in range(min(_NBUF, _NCHUNK)):
        gathers[c] = start_gather(c)
    for c in range(_NCHUNK):
        gathers[c].wait()
        scatters[c] = pltpu.async_copy(
            bufs[c % _NBUF], out_hbm.at[pl.ds(base + c * _CHUNK, _CHUNK)],
            sem_s[c % _NBUF],
        )
        nxt = c + _NBUF
        if nxt < _NCHUNK:
            scatters[nxt - _NBUF].wait()
            gathers[nxt] = start_gather(nxt)
    for c in range(max(0, _NCHUNK - _NBUF), _NCHUNK):
        scatters[c].wait()


@functools.cache
def _gather_call():
    return pl.kernel(
        _gather_body,
        out_type=jax.ShapeDtypeStruct((NTOK, D), jnp.float32),
        mesh=plsc.VectorSubcoreMesh(core_axis_name="c", subcore_axis_name="s"),
        scratch_types=[
            pltpu.VMEM((_ROWS_PER_W,), jnp.int32),
            pltpu.VMEM((_CHUNK, D), jnp.float32),
            pltpu.VMEM((_CHUNK, D), jnp.float32),
            pltpu.VMEM((_CHUNK, D), jnp.float32),
            pltpu.SemaphoreType.DMA,
            pltpu.SemaphoreType.DMA,
            pltpu.SemaphoreType.DMA,
            pltpu.SemaphoreType.DMA,
            pltpu.SemaphoreType.DMA,
            pltpu.SemaphoreType.DMA,
        ],
    )


def kernel(x, W1, b1, W2, b2, codebook):
    r = jnp.transpose(x, (0, 2, 1)).reshape(B, L4, 4 * C_IN).astype(_BF)
    w1c = jnp.transpose(W1, (2, 1, 0)).reshape(4 * C_IN, HID).astype(_BF)
    w2c = jnp.transpose(W2, (2, 1, 0)).reshape(4 * HID, D).astype(_BF)
    b1r = b1.reshape(1, HID)
    b2r = b2.reshape(1, D)
    cbt = codebook.T.astype(_BF)

    z_tok = _conv_call(r, w1c, b1r, w2c, b2r)
    csq = _csq_call(codebook)
    z_flat = z_tok.reshape(NTOK, D)
    idx2d = _vq_call(z_flat, cbt, csq)
    indices = idx2d.reshape(NTOK)
    zq_flat = _gather_call()(codebook, indices)

    z_q = zq_flat.reshape(B, D, L4)
    z_e = jnp.transpose(z_tok, (0, 2, 1))
    return (z_q, indices, z_e)


# speedup vs baseline: 1.5213x; 1.5213x over previous
import functools

import jax
import jax.numpy as jnp
from jax import lax
from jax.experimental import pallas as pl
from jax.experimental.pallas import tpu as pltpu
from jax.experimental.pallas import tpu_sc as plsc

B, C_IN, L = 16, 128, 4096
HID, D, K = 384, 256, 8192
L2, L4 = L // 2, L // 4
NTOK = B * L4

TT = 1024
KK = 1024
NT_T = NTOK // TT
NT_K = K // KK

_BF = jnp.bfloat16
_F32 = jnp.float32


def _conv_body(r_ref, w1_ref, b1_ref, w2_ref, b2_ref, out_ref):
    r = r_ref[0]
    zs128 = jnp.zeros((1, C_IN), _BF)
    xm1 = jnp.concatenate([zs128, r[:-1, 3 * C_IN:4 * C_IN]], axis=0)
    a_e = jnp.concatenate([xm1, r[:, 0:3 * C_IN]], axis=1)
    xp4 = jnp.concatenate([r[1:, 0:C_IN], zs128], axis=0)
    a_o = jnp.concatenate([r[:, C_IN:4 * C_IN], xp4], axis=1)
    w1 = w1_ref[...]
    b1 = b1_ref[...]
    z1e = jnp.dot(a_e, w1, preferred_element_type=_F32) + b1
    z1o = jnp.dot(a_o, w1, preferred_element_type=_F32) + b1
    z1e_b = z1e.astype(_BF)
    z1o_b = z1o.astype(_BF)
    zs384 = jnp.zeros((1, HID), _BF)
    sd_o = jnp.concatenate([zs384, z1o_b[:-1]], axis=0)
    su_e = jnp.concatenate([z1e_b[1:], zs384], axis=0)
    w2 = w2_ref[...]
    ze = jnp.dot(sd_o, w2[0:HID], preferred_element_type=_F32)
    ze = ze + jnp.dot(z1e_b, w2[HID:2 * HID], preferred_element_type=_F32)
    ze = ze + jnp.dot(z1o_b, w2[2 * HID:3 * HID], preferred_element_type=_F32)
    ze = ze + jnp.dot(su_e, w2[3 * HID:4 * HID], preferred_element_type=_F32)
    ze = ze + b2_ref[...]
    out_ref[0] = ze


def _conv_call(r, w1c, b1, w2c, b2):
    return pl.pallas_call(
        _conv_body,
        grid=(B,),
        in_specs=[
            pl.BlockSpec((1, L4, 4 * C_IN), lambda b: (b, 0, 0)),
            pl.BlockSpec((4 * C_IN, HID), lambda b: (0, 0)),
            pl.BlockSpec((1, HID), lambda b: (0, 0)),
            pl.BlockSpec((4 * HID, D), lambda b: (0, 0)),
            pl.BlockSpec((1, D), lambda b: (0, 0)),
        ],
        out_specs=pl.BlockSpec((1, L4, D), lambda b: (b, 0, 0)),
        out_shape=jax.ShapeDtypeStruct((B, L4, D), jnp.float32),
    )(r, w1c, b1, w2c, b2)


def _csq_body(cb_ref, out_ref):
    cb = cb_ref[...]
    s = jnp.sum(cb * cb, axis=1)
    out_ref[...] = s.reshape(1, KK)


def _csq_call(cb):
    return pl.pallas_call(
        _csq_body,
        grid=(NT_K,),
        in_specs=[pl.BlockSpec((KK, D), lambda j: (j, 0))],
        out_specs=pl.BlockSpec((1, KK), lambda j: (0, j)),
        out_shape=jax.ShapeDtypeStruct((1, K), jnp.float32),
    )(cb)


_TILES_PER_CHUNK = 4096 // KK


def _vq_body(z_ref, cbt_ref, csq_ref, idx_ref, zsq_s, zm2_s, cmin_s, cidx_s, gmin_s, gidx_s):
    j = pl.program_id(1)

    @pl.when(j == 0)
    def _init():
        z = z_ref[...]
        zsq_s[...] = jnp.sum(z * z, axis=1, keepdims=True)
        zm2_s[...] = (-2.0 * z).astype(_BF)
        gmin_s[...] = jnp.full((TT, 1), jnp.inf, jnp.float32)
        gidx_s[...] = jnp.zeros((TT, 1), jnp.int32)

    @pl.when(j % _TILES_PER_CHUNK == 0)
    def _chunk_init():
        cmin_s[...] = jnp.full((TT, 1), jnp.inf, jnp.float32)
        cidx_s[...] = jnp.zeros((TT, 1), jnp.int32)

    mm2 = jnp.dot(zm2_s[...], cbt_ref[...], preferred_element_type=_F32)
    dist = (zsq_s[...] + mm2) + csq_ref[...]
    tmin = jnp.min(dist, axis=1, keepdims=True)
    lanes = lax.broadcasted_iota(jnp.int32, (TT, KK), 1)
    tidx = jnp.min(jnp.where(dist == tmin, lanes, K), axis=1, keepdims=True)
    tidx = tidx + j * KK
    better = tmin < cmin_s[...]
    cmin_s[...] = jnp.where(better, tmin, cmin_s[...])
    cidx_s[...] = jnp.where(better, tidx, cidx_s[...])

    @pl.when(j % _TILES_PER_CHUNK == _TILES_PER_CHUNK - 1)
    def _commit():
        take = cmin_s[...] < gmin_s[...]
        gmin_s[...] = jnp.where(take, cmin_s[...], gmin_s[...]).astype(_BF).astype(_F32)
        gidx_s[...] = jnp.where(take, cidx_s[...], gidx_s[...])

    @pl.when(j == NT_K - 1)
    def _out():
        idx_ref[...] = gidx_s[...]


def _vq_call(z_flat, cbt, csq):
    return pl.pallas_call(
        _vq_body,
        grid=(NT_T, NT_K),
        in_specs=[
            pl.BlockSpec((TT, D), lambda i, j: (i, 0)),
            pl.BlockSpec((D, KK), lambda i, j: (0, j)),
            pl.BlockSpec((1, KK), lambda i, j: (0, j)),
        ],
        out_specs=pl.BlockSpec((TT, 1), lambda i, j: (i, 0)),
        out_shape=jax.ShapeDtypeStruct((NTOK, 1), jnp.int32),
        scratch_shapes=[
            pltpu.VMEM((TT, 1), jnp.float32),
            pltpu.VMEM((TT, D), _BF),
            pltpu.VMEM((TT, 1), jnp.float32),
            pltpu.VMEM((TT, 1), jnp.int32),
            pltpu.VMEM((TT, 1), jnp.float32),
            pltpu.VMEM((TT, 1), jnp.int32),
        ],
    )(z_flat, cbt, csq)


_SC_NW = 32
_SC_TOK = 4096
_SC_RPW = _SC_TOK // _SC_NW
_TC_TOK = NTOK - _SC_TOK
_TC_TILE = 1024


def _gather_body(cb_hbm, idx_hbm, out_hbm, idx_v, rows_v, sem_g, sem_s):
    wid = lax.axis_index("s") * 2 + lax.axis_index("c")
    base = wid * _SC_RPW
    pltpu.sync_copy(idx_hbm.at[pl.ds(base, _SC_RPW)], idx_v)
    pltpu.async_copy(cb_hbm.at[idx_v], rows_v, sem_g).wait()
    pltpu.async_copy(rows_v, out_hbm.at[pl.ds(base, _SC_RPW)], sem_s).wait()


@functools.cache
def _gather_call():
    return pl.kernel(
        _gather_body,
        out_type=jax.ShapeDtypeStruct((_SC_TOK, D), jnp.float32),
        mesh=plsc.VectorSubcoreMesh(core_axis_name="c", subcore_axis_name="s"),
        scratch_types=[
            pltpu.VMEM((_SC_RPW,), jnp.int32),
            pltpu.VMEM((_SC_RPW, D), jnp.float32),
            pltpu.SemaphoreType.DMA,
            pltpu.SemaphoreType.DMA,
        ],
    )


def _tc_gather_body(idx_ref, cb_ref, out_ref):
    def body(r, carry):
        k = idx_ref[r]
        out_ref[pl.ds(r, 1), :] = cb_ref[pl.ds(k, 1), :]
        return carry

    lax.fori_loop(0, _TC_TILE, body, 0)


def _tc_gather_call(cb, idx_tc):
    return pl.pallas_call(
        _tc_gather_body,
        grid=(_TC_TOK // _TC_TILE,),
        in_specs=[
            pl.BlockSpec((_TC_TILE,), lambda i: (i,), memory_space=pltpu.SMEM),
            pl.BlockSpec((K, D), lambda i: (0, 0)),
        ],
        out_specs=pl.BlockSpec((_TC_TILE, D), lambda i: (i, 0)),
        out_shape=jax.ShapeDtypeStruct((_TC_TOK, D), jnp.float32),
    )(idx_tc, cb)


def kernel(x, W1, b1, W2, b2, codebook):
    r = jnp.transpose(x, (0, 2, 1)).reshape(B, L4, 4 * C_IN).astype(_BF)
    w1c = jnp.transpose(W1, (2, 1, 0)).reshape(4 * C_IN, HID).astype(_BF)
    w2c = jnp.transpose(W2, (2, 1, 0)).reshape(4 * HID, D).astype(_BF)
    b1r = b1.reshape(1, HID)
    b2r = b2.reshape(1, D)
    cbt = codebook.T.astype(_BF)

    z_tok = _conv_call(r, w1c, b1r, w2c, b2r)
    csq = _csq_call(codebook)
    z_flat = z_tok.reshape(NTOK, D)
    idx2d = _vq_call(z_flat, cbt, csq)
    indices = idx2d.reshape(NTOK)
    zq_sc = _gather_call()(codebook, indices[:_SC_TOK])
    zq_tc = _tc_gather_call(codebook, indices[_SC_TOK:])
    zq_flat = jnp.concatenate([zq_sc, zq_tc], axis=0)

    z_q = zq_flat.reshape(B, D, L4)
    z_e = jnp.transpose(z_tok, (0, 2, 1))
    return (z_q, indices, z_e)
